# Initial kernel scaffold; baseline (speedup 1.0000x reference)
import jax
import jax.numpy as jnp
from jax.experimental import pallas as pl


def kernel(data, structure, W, b):
    raise NotImplementedError("write your pallas kernel here")


# speedup vs baseline: 4.2985x; 4.2985x over previous
import functools

import jax
import jax.numpy as jnp
from jax import lax
from jax.experimental import pallas as pl
from jax.experimental.pallas import tpu as pltpu
from jax.experimental.pallas import tpu_sc as plsc

N = 10000
E = 160000
D = 256
H = 128
DEPTH = 3

NC = 2
NS = 16
CHUNK = 64
EPT = 10240
E_PAD = NS * EPT
NCH = EPT // CHUNK
NSEG = 2
SEGC = NCH // NSEG
NACC = 10240
RB = 1000
GRID_I = N // RB

_sc_mesh = plsc.VectorSubcoreMesh(core_axis_name="c", subcore_axis_name="s")


def _segsum_body(x_hbm, srcs_hbm, dst_hbm, g_hbm,
                 sbuf, dbuf, rb0, rb1, acc, sem0, sem1):
    c = lax.axis_index("c")
    s = lax.axis_index("s")

    @pl.loop(0, CHUNK)
    def _zero_rows(i):
        for k in range(H // 16):
            rb0[i, pl.ds(k * 16, 16)] = jnp.zeros((16,), jnp.float32)

    @pl.loop(0, NACC // NS // CHUNK)
    def _zero_acc(k):
        pltpu.sync_copy(rb0, acc.at[pl.ds(s * (NACC // NS) + k * CHUNK, CHUNK)])

    plsc.subcore_barrier()

    @pl.loop(0, NSEG)
    def _segment(seg):
        pltpu.sync_copy(
            srcs_hbm.at[pl.ds(c * (E_PAD // CHUNK) + s * NCH + seg * SEGC, SEGC)],
            sbuf)
        pltpu.sync_copy(dst_hbm.at[pl.ds(s * NCH + seg * SEGC, SEGC)], dbuf)

        pltpu.async_copy(x_hbm.at[sbuf.at[0]], rb0, sem0)
        pltpu.async_copy(x_hbm.at[sbuf.at[1]], rb1, sem1)

        @pl.loop(0, SEGC // 2)
        def _main(jo):
            for b, (rb, sem) in enumerate(((rb0, sem0), (rb1, sem1))):
                j = jo * 2 + b
                pltpu.make_async_copy(x_hbm.at[sbuf.at[j]], rb, sem).wait()
                pltpu.sync_copy(rb, acc.at[dbuf.at[j]], add=True)
                jn = jnp.minimum(j + 2, SEGC - 1)
                pltpu.async_copy(x_hbm.at[sbuf.at[jn]], rb, sem)

        pltpu.make_async_copy(x_hbm.at[sbuf.at[0]], rb0, sem0).wait()
        pltpu.make_async_copy(x_hbm.at[sbuf.at[0]], rb1, sem1).wait()

    plsc.subcore_barrier()

    @pl.when(s < NS - 1)
    def _flush_main():
        pltpu.sync_copy(acc.at[pl.ds(s * 632, 632)],
                        g_hbm.at[pl.ds(c * N + s * 632, 632)])

    @pl.when(s == NS - 1)
    def _flush_tail():
        pltpu.sync_copy(acc.at[pl.ds((NS - 1) * 632, N - (NS - 1) * 632)],
                        g_hbm.at[pl.ds(c * N + (NS - 1) * 632, N - (NS - 1) * 632)])


_segsum = functools.partial(
    pl.kernel,
    out_type=jax.ShapeDtypeStruct((2 * N, H), jnp.float32),
    mesh=_sc_mesh,
    scratch_types=[
        pltpu.VMEM((SEGC, CHUNK), jnp.int32),
        pltpu.VMEM((SEGC, CHUNK), jnp.int32),
        pltpu.VMEM((CHUNK, H), jnp.float32),
        pltpu.VMEM((CHUNK, H), jnp.float32),
        pltpu.VMEM_SHARED((NACC, H), jnp.float32),
        pltpu.SemaphoreType.DMA,
        pltpu.SemaphoreType.DMA,
    ],
)(_segsum_body)


def _counts_body(dst_hbm, cnt_hbm, dbuf, buf, acc):
    c = lax.axis_index("c")
    s = lax.axis_index("s")

    @pl.when(c == 0)
    def _():
        pltpu.sync_copy(dst_hbm.at[pl.ds(s * NCH, NCH)], dbuf)

        @pl.loop(0, CHUNK)
        def _fill_zer(i):
            for k in range(H // 16):
                buf[i, pl.ds(k * 16, 16)] = jnp.zeros((16,), jnp.float32)

        @pl.loop(0, NACC // NS // CHUNK)
        def _zero_acc(k):
            pltpu.sync_copy(buf, acc.at[pl.ds(s * (NACC // NS) + k * CHUNK, CHUNK)])

        @pl.loop(0, CHUNK)
        def _fill_ones(i):
            for k in range(H // 16):
                buf[i, pl.ds(k * 16, 16)] = jnp.ones((16,), jnp.float32)

        plsc.subcore_barrier()

        @pl.loop(0, NCH)
        def _count(j):
            pltpu.sync_copy(buf, acc.at[dbuf.at[j]], add=True)

        plsc.subcore_barrier()
        pltpu.sync_copy(acc.at[pl.ds(s * (NACC // NS), NACC // NS)],
                        cnt_hbm.at[pl.ds(s * (NACC // NS), NACC // NS)])


_counts = functools.partial(
    pl.kernel,
    out_type=jax.ShapeDtypeStruct((NACC, H), jnp.float32),
    mesh=_sc_mesh,
    scratch_types=[
        pltpu.VMEM((NCH, CHUNK), jnp.int32),
        pltpu.VMEM((CHUNK, H), jnp.float32),
        pltpu.VMEM_SHARED((NACC, H), jnp.float32),
    ],
)(_counts_body)


def _mm_body(x_ref, w_ref, b_ref, o_ref):
    x = x_ref[...]
    w = w_ref[...]
    o = lax.dot_general(x, w, (((1,), (1,)), ((), ())),
                        precision=lax.Precision.HIGHEST,
                        preferred_element_type=jnp.float32)
    o_ref[...] = o + b_ref[...][None, :]


def _matmul(data, W, b):
    return pl.pallas_call(
        _mm_body,
        grid=(GRID_I, 2),
        in_specs=[
            pl.BlockSpec((RB, D), lambda i, h: (i, 0)),
            pl.BlockSpec((H, D), lambda i, h: (h, 0)),
            pl.BlockSpec((H,), lambda i, h: (h,)),
        ],
        out_specs=pl.BlockSpec((RB, H), lambda i, h: (h * GRID_I + i, 0)),
        out_shape=jax.ShapeDtypeStruct((2 * N, H), jnp.float32),
    )(data, W, b)


def _lap(x, g, cnt_ref):
    cnt = cnt_ref[...][:, 0:1]
    mask = cnt > 0.0
    invc = 1.0 / jnp.where(mask, cnt, 1.0)
    return jnp.where(mask, x - g * invc, 0.0)


def _combine_first_body(x_ref, g_ref, cnt_ref, y_ref, o_ref):
    x = x_ref[...]
    y = _lap(x, g_ref[...], cnt_ref)
    y_ref[...] = y
    o_ref[...] = y + x


def _combine_body(x_ref, g_ref, p_ref, a_ref, cnt_ref, y_ref, o_ref):
    y = 2.0 * _lap(x_ref[...], g_ref[...], cnt_ref) - p_ref[...]
    y_ref[...] = y
    o_ref[...] = a_ref[...] + y


def _final_body(x_ref, g_ref, p_ref, a_ref, cnt_ref, o_ref):
    y = 2.0 * _lap(x_ref[...], g_ref[...], cnt_ref) - p_ref[...]
    o_ref[...] = jnp.maximum(a_ref[...] + y, 0.0)


_cat_spec = pl.BlockSpec((RB, H), lambda i, h: (h * GRID_I + i, 0))
_cnt_spec = pl.BlockSpec((RB, H), lambda i, h: (i, 0))
_cat_struct = jax.ShapeDtypeStruct((2 * N, H), jnp.float32)


def _combine_first(x, g, cnt):
    return pl.pallas_call(
        _combine_first_body,
        grid=(GRID_I, 2),
        in_specs=[_cat_spec, _cat_spec, _cnt_spec],
        out_specs=[_cat_spec, _cat_spec],
        out_shape=[_cat_struct, _cat_struct],
    )(x, g, cnt)


def _combine(x, g, prev, out, cnt):
    return pl.pallas_call(
        _combine_body,
        grid=(GRID_I, 2),
        in_specs=[_cat_spec, _cat_spec, _cat_spec, _cat_spec, _cnt_spec],
        out_specs=[_cat_spec, _cat_spec],
        out_shape=[_cat_struct, _cat_struct],
    )(x, g, prev, out, cnt)


def _final(x, g, prev, out, cnt):
    return pl.pallas_call(
        _final_body,
        grid=(GRID_I, 2),
        in_specs=[_cat_spec, _cat_spec, _cat_spec, _cat_spec, _cnt_spec],
        out_specs=pl.BlockSpec((RB, H), lambda i, h: (i, h)),
        out_shape=jax.ShapeDtypeStruct((N, D), jnp.float32),
    )(x, g, prev, out, cnt)


def kernel(data, structure, W, b):
    src = structure[0]
    dst = structure[1]

    pad = E_PAD - E
    src_p = jnp.concatenate([src, jnp.zeros((pad,), jnp.int32)])
    dst_p = jnp.concatenate([dst, jnp.full((pad,), N, jnp.int32)])
    srcs2 = jnp.concatenate([src_p, src_p + N]).reshape(2 * (E_PAD // CHUNK), CHUNK)
    dst2 = dst_p.reshape(E_PAD // CHUNK, CHUNK)

    cnt = _counts(dst2)

    o2 = _matmul(data, W, b)
    g = _segsum(o2, srcs2, dst2)
    o1, out = _combine_first(o2, g, cnt)

    g = _segsum(o1, srcs2, dst2)
    o1b, out = _combine(o1, g, o2, out, cnt)

    g = _segsum(o1b, srcs2, dst2)
    o1c, out = _combine(o1b, g, o1, out, cnt)

    g = _segsum(o1c, srcs2, dst2)
    return _final(o1c, g, o1b, out, cnt)


# speedup vs baseline: 4.4298x; 1.0306x over previous
import functools

import jax
import jax.numpy as jnp
from jax import lax
from jax.experimental import pallas as pl
from jax.experimental.pallas import tpu as pltpu
from jax.experimental.pallas import tpu_sc as plsc

N = 10000
E = 160000
D = 256
H = 128
DEPTH = 3

NC = 2
NS = 16
CHUNK = 64
EPT = 10240
E_PAD = NS * EPT
NCH = EPT // CHUNK
NSEG = 4
SEGC = NCH // NSEG
NACC = 10240
RB = 1000
GRID_I = N // RB

_sc_mesh = plsc.VectorSubcoreMesh(core_axis_name="c", subcore_axis_name="s")


def _segsum_body(x_hbm, srcs_hbm, dst_hbm, g_hbm,
                 sbuf, dbuf, rb0, rb1, rb2, rb3, acc,
                 sg0, sg1, sg2, sg3, ss0, ss1, ss2, ss3):
    c = lax.axis_index("c")
    s = lax.axis_index("s")
    rbs = (rb0, rb1, rb2, rb3)
    sgs = (sg0, sg1, sg2, sg3)
    sss = (ss0, ss1, ss2, ss3)

    @pl.loop(0, CHUNK)
    def _zero_rows(i):
        for k in range(H // 16):
            rb0[i, pl.ds(k * 16, 16)] = jnp.zeros((16,), jnp.float32)

    @pl.loop(0, NACC // NS // CHUNK)
    def _zero_acc(k):
        pltpu.sync_copy(rb0, acc.at[pl.ds(s * (NACC // NS) + k * CHUNK, CHUNK)])

    plsc.subcore_barrier()

    def _g_start(j, b):
        pltpu.async_copy(x_hbm.at[sbuf.at[j]], rbs[b], sgs[b])

    def _g_wait(b):
        pltpu.make_async_copy(x_hbm.at[sbuf.at[0]], rbs[b], sgs[b]).wait()

    def _s_start(j, b):
        pltpu.async_copy(rbs[b], acc.at[dbuf.at[j]], sss[b], add=True)

    def _s_wait(b):
        pltpu.make_async_copy(rbs[b], acc.at[dbuf.at[0]], sss[b]).wait()

    @pl.loop(0, NSEG)
    def _segment(seg):
        pltpu.sync_copy(
            srcs_hbm.at[pl.ds(c * (E_PAD // CHUNK) + s * NCH + seg * SEGC, SEGC)],
            sbuf)
        pltpu.sync_copy(dst_hbm.at[pl.ds(s * NCH + seg * SEGC, SEGC)], dbuf)

        _g_start(0, 0)
        _g_start(1, 1)
        _g_wait(0); _s_start(0, 0); _g_start(2, 2)
        _g_wait(1); _s_start(1, 1); _g_start(3, 3)

        @pl.loop(0, (SEGC - 4) // 4)
        def _main(jo):
            for db in range(4):
                j = 2 + jo * 4 + db
                b = (2 + db) % 4
                bq = db
                _g_wait(b)
                _s_start(j, b)
                _s_wait(bq)
                _g_start(j + 2, bq)

        _g_wait(2); _s_start(SEGC - 2, 2); _s_wait(0)
        _g_wait(3); _s_start(SEGC - 1, 3); _s_wait(1)
        _s_wait(2)
        _s_wait(3)

    plsc.subcore_barrier()

    @pl.when(s < NS - 1)
    def _flush_main():
        pltpu.sync_copy(acc.at[pl.ds(s * 632, 632)],
                        g_hbm.at[pl.ds(c * N + s * 632, 632)])

    @pl.when(s == NS - 1)
    def _flush_tail():
        pltpu.sync_copy(acc.at[pl.ds((NS - 1) * 632, N - (NS - 1) * 632)],
                        g_hbm.at[pl.ds(c * N + (NS - 1) * 632, N - (NS - 1) * 632)])


_segsum = functools.partial(
    pl.kernel,
    out_type=jax.ShapeDtypeStruct((2 * N, H), jnp.float32),
    mesh=_sc_mesh,
    scratch_types=[
        pltpu.VMEM((SEGC, CHUNK), jnp.int32),
        pltpu.VMEM((SEGC, CHUNK), jnp.int32),
        pltpu.VMEM((CHUNK, H), jnp.float32),
        pltpu.VMEM((CHUNK, H), jnp.float32),
        pltpu.VMEM((CHUNK, H), jnp.float32),
        pltpu.VMEM((CHUNK, H), jnp.float32),
        pltpu.VMEM_SHARED((NACC, H), jnp.float32),
    ] + [pltpu.SemaphoreType.DMA] * 8,
)(_segsum_body)


def _counts_body(dst_hbm, cnt_hbm, dbuf, buf, acc):
    c = lax.axis_index("c")
    s = lax.axis_index("s")

    @pl.when(c == 0)
    def _():
        pltpu.sync_copy(dst_hbm.at[pl.ds(s * NCH, NCH)], dbuf)

        @pl.loop(0, CHUNK)
        def _fill_zer(i):
            for k in range(H // 16):
                buf[i, pl.ds(k * 16, 16)] = jnp.zeros((16,), jnp.float32)

        @pl.loop(0, NACC // NS // CHUNK)
        def _zero_acc(k):
            pltpu.sync_copy(buf, acc.at[pl.ds(s * (NACC // NS) + k * CHUNK, CHUNK)])

        @pl.loop(0, CHUNK)
        def _fill_ones(i):
            for k in range(H // 16):
                buf[i, pl.ds(k * 16, 16)] = jnp.ones((16,), jnp.float32)

        plsc.subcore_barrier()

        @pl.loop(0, NCH)
        def _count(j):
            pltpu.sync_copy(buf, acc.at[dbuf.at[j]], add=True)

        plsc.subcore_barrier()
        pltpu.sync_copy(acc.at[pl.ds(s * (NACC // NS), NACC // NS)],
                        cnt_hbm.at[pl.ds(s * (NACC // NS), NACC // NS)])


_counts = functools.partial(
    pl.kernel,
    out_type=jax.ShapeDtypeStruct((NACC, H), jnp.float32),
    mesh=_sc_mesh,
    scratch_types=[
        pltpu.VMEM((NCH, CHUNK), jnp.int32),
        pltpu.VMEM((CHUNK, H), jnp.float32),
        pltpu.VMEM_SHARED((NACC, H), jnp.float32),
    ],
)(_counts_body)


def _mm_body(x_ref, w_ref, b_ref, o_ref):
    x = x_ref[...]
    w = w_ref[...]
    o = lax.dot_general(x, w, (((1,), (1,)), ((), ())),
                        precision=lax.Precision.HIGHEST,
                        preferred_element_type=jnp.float32)
    o_ref[...] = o + b_ref[...][None, :]


def _matmul(data, W, b):
    return pl.pallas_call(
        _mm_body,
        grid=(GRID_I, 2),
        in_specs=[
            pl.BlockSpec((RB, D), lambda i, h: (i, 0)),
            pl.BlockSpec((H, D), lambda i, h: (h, 0)),
            pl.BlockSpec((H,), lambda i, h: (h,)),
        ],
        out_specs=pl.BlockSpec((RB, H), lambda i, h: (h * GRID_I + i, 0)),
        out_shape=jax.ShapeDtypeStruct((2 * N, H), jnp.float32),
    )(data, W, b)


def _lap(x, g, cnt_ref):
    cnt = cnt_ref[...][:, 0:1]
    mask = cnt > 0.0
    invc = 1.0 / jnp.where(mask, cnt, 1.0)
    return jnp.where(mask, x - g * invc, 0.0)


def _combine_first_body(x_ref, g_ref, cnt_ref, y_ref, o_ref):
    x = x_ref[...]
    y = _lap(x, g_ref[...], cnt_ref)
    y_ref[...] = y
    o_ref[...] = y + x


def _combine_body(x_ref, g_ref, p_ref, a_ref, cnt_ref, y_ref, o_ref):
    y = 2.0 * _lap(x_ref[...], g_ref[...], cnt_ref) - p_ref[...]
    y_ref[...] = y
    o_ref[...] = a_ref[...] + y


def _final_body(x_ref, g_ref, p_ref, a_ref, cnt_ref, o_ref):
    y = 2.0 * _lap(x_ref[...], g_ref[...], cnt_ref) - p_ref[...]
    o_ref[...] = jnp.maximum(a_ref[...] + y, 0.0)


_cat_spec = pl.BlockSpec((RB, H), lambda i, h: (h * GRID_I + i, 0))
_cnt_spec = pl.BlockSpec((RB, H), lambda i, h: (i, 0))
_cat_struct = jax.ShapeDtypeStruct((2 * N, H), jnp.float32)


def _combine_first(x, g, cnt):
    return pl.pallas_call(
        _combine_first_body,
        grid=(GRID_I, 2),
        in_specs=[_cat_spec, _cat_spec, _cnt_spec],
        out_specs=[_cat_spec, _cat_spec],
        out_shape=[_cat_struct, _cat_struct],
    )(x, g, cnt)


def _combine(x, g, prev, out, cnt):
    return pl.pallas_call(
        _combine_body,
        grid=(GRID_I, 2),
        in_specs=[_cat_spec, _cat_spec, _cat_spec, _cat_spec, _cnt_spec],
        out_specs=[_cat_spec, _cat_spec],
        out_shape=[_cat_struct, _cat_struct],
    )(x, g, prev, out, cnt)


def _final(x, g, prev, out, cnt):
    return pl.pallas_call(
        _final_body,
        grid=(GRID_I, 2),
        in_specs=[_cat_spec, _cat_spec, _cat_spec, _cat_spec, _cnt_spec],
        out_specs=pl.BlockSpec((RB, H), lambda i, h: (i, h)),
        out_shape=jax.ShapeDtypeStruct((N, D), jnp.float32),
    )(x, g, prev, out, cnt)


def kernel(data, structure, W, b):
    src = structure[0]
    dst = structure[1]

    pad = E_PAD - E
    src_p = jnp.concatenate([src, jnp.zeros((pad,), jnp.int32)])
    dst_p = jnp.concatenate([dst, jnp.full((pad,), N, jnp.int32)])
    srcs2 = jnp.concatenate([src_p, src_p + N]).reshape(2 * (E_PAD // CHUNK), CHUNK)
    dst2 = dst_p.reshape(E_PAD // CHUNK, CHUNK)

    cnt = _counts(dst2)

    o2 = _matmul(data, W, b)
    g = _segsum(o2, srcs2, dst2)
    o1, out = _combine_first(o2, g, cnt)

    g = _segsum(o1, srcs2, dst2)
    o1b, out = _combine(o1, g, o2, out, cnt)

    g = _segsum(o1b, srcs2, dst2)
    o1c, out = _combine(o1b, g, o1, out, cnt)

    g = _segsum(o1c, srcs2, dst2)
    return _final(o1c, g, o1b, out, cnt)


# speedup vs baseline: 4.4307x; 1.0002x over previous
import functools

import jax
import jax.numpy as jnp
from jax import lax
from jax.experimental import pallas as pl
from jax.experimental.pallas import tpu as pltpu
from jax.experimental.pallas import tpu_sc as plsc

N = 10000
E = 160000
D = 256
H = 128
DEPTH = 3

NC = 2
NS = 16
CHUNK = 64
EPT = 10240
E_PAD = NS * EPT
NCH = EPT // CHUNK
NSEG = 4
SEGC = NCH // NSEG
NACC = 10240
RB = 1000
GRID_I = N // RB

_sc_mesh = plsc.VectorSubcoreMesh(core_axis_name="c", subcore_axis_name="s")


def _segsum_body(x_hbm, srcs_hbm, dst_hbm, g_hbm,
                 sbuf, dbuf, rb0, rb1, rb2, rb3, acc,
                 sg0, sg1, sg2, sg3, ss0, ss1, ss2, ss3):
    c = lax.axis_index("c")
    s = lax.axis_index("s")
    rbs = (rb0, rb1, rb2, rb3)
    sgs = (sg0, sg1, sg2, sg3)
    sss = (ss0, ss1, ss2, ss3)

    @pl.loop(0, CHUNK)
    def _zero_rows(i):
        for k in range(H // 16):
            rb0[i, pl.ds(k * 16, 16)] = jnp.zeros((16,), jnp.float32)

    @pl.loop(0, NACC // NS // CHUNK)
    def _zero_acc(k):
        pltpu.sync_copy(rb0, acc.at[pl.ds(s * (NACC // NS) + k * CHUNK, CHUNK)])

    plsc.subcore_barrier()

    def _g_start(j, b):
        pltpu.async_copy(x_hbm.at[sbuf.at[j]], rbs[b], sgs[b])

    def _g_wait(b):
        pltpu.make_async_copy(x_hbm.at[sbuf.at[0]], rbs[b], sgs[b]).wait()

    def _s_start(j, b):
        pltpu.async_copy(rbs[b], acc.at[dbuf.at[j]], sss[b], add=True)

    def _s_wait(b):
        pltpu.make_async_copy(rbs[b], acc.at[dbuf.at[0]], sss[b]).wait()

    @pl.loop(0, NSEG)
    def _segment(seg):
        pltpu.sync_copy(
            srcs_hbm.at[pl.ds(c * (E_PAD // CHUNK) + s * NCH + seg * SEGC, SEGC)],
            sbuf)
        pltpu.sync_copy(dst_hbm.at[pl.ds(s * NCH + seg * SEGC, SEGC)], dbuf)

        _g_start(0, 0)
        _g_start(1, 1)
        _g_wait(0); _s_start(0, 0); _g_start(2, 2)
        _g_wait(1); _s_start(1, 1); _g_start(3, 3)

        @pl.loop(0, (SEGC - 4) // 4)
        def _main(jo):
            for db in range(4):
                j = 2 + jo * 4 + db
                b = (2 + db) % 4
                bq = db
                _g_wait(b)
                _s_start(j, b)
                _s_wait(bq)
                _g_start(j + 2, bq)

        _g_wait(2); _s_start(SEGC - 2, 2); _s_wait(0)
        _g_wait(3); _s_start(SEGC - 1, 3); _s_wait(1)
        _s_wait(2)
        _s_wait(3)

    plsc.subcore_barrier()

    @pl.when(s < NS - 1)
    def _flush_main():
        pltpu.sync_copy(acc.at[pl.ds(s * 632, 632)],
                        g_hbm.at[pl.ds(c * N + s * 632, 632)])

    @pl.when(s == NS - 1)
    def _flush_tail():
        pltpu.sync_copy(acc.at[pl.ds((NS - 1) * 632, N - (NS - 1) * 632)],
                        g_hbm.at[pl.ds(c * N + (NS - 1) * 632, N - (NS - 1) * 632)])


_segsum = functools.partial(
    pl.kernel,
    out_type=jax.ShapeDtypeStruct((2 * N, H), jnp.float32),
    mesh=_sc_mesh,
    scratch_types=[
        pltpu.VMEM((SEGC, CHUNK), jnp.int32),
        pltpu.VMEM((SEGC, CHUNK), jnp.int32),
        pltpu.VMEM((CHUNK, H), jnp.float32),
        pltpu.VMEM((CHUNK, H), jnp.float32),
        pltpu.VMEM((CHUNK, H), jnp.float32),
        pltpu.VMEM((CHUNK, H), jnp.float32),
        pltpu.VMEM_SHARED((NACC, H), jnp.float32),
    ] + [pltpu.SemaphoreType.DMA] * 8,
)(_segsum_body)


def _counts_body(dst_hbm, cnt_hbm, dbuf, buf, acc):
    c = lax.axis_index("c")
    s = lax.axis_index("s")

    @pl.when(c == 0)
    def _():
        pltpu.sync_copy(dst_hbm.at[pl.ds(s * NCH, NCH)], dbuf)

        @pl.loop(0, CHUNK)
        def _fill_zer(i):
            for k in range(H // 16):
                buf[i, pl.ds(k * 16, 16)] = jnp.zeros((16,), jnp.float32)

        @pl.loop(0, NACC // NS // CHUNK)
        def _zero_acc(k):
            pltpu.sync_copy(buf, acc.at[pl.ds(s * (NACC // NS) + k * CHUNK, CHUNK)])

        @pl.loop(0, CHUNK)
        def _fill_ones(i):
            for k in range(H // 16):
                buf[i, pl.ds(k * 16, 16)] = jnp.ones((16,), jnp.float32)

        plsc.subcore_barrier()

        @pl.loop(0, NCH)
        def _count(j):
            pltpu.sync_copy(buf, acc.at[dbuf.at[j]], add=True)

        plsc.subcore_barrier()
        pltpu.sync_copy(acc.at[pl.ds(s * (NACC // NS), NACC // NS)],
                        cnt_hbm.at[pl.ds(s * (NACC // NS), NACC // NS)])


_counts = functools.partial(
    pl.kernel,
    out_type=jax.ShapeDtypeStruct((NACC, H), jnp.float32),
    mesh=_sc_mesh,
    scratch_types=[
        pltpu.VMEM((NCH, CHUNK), jnp.int32),
        pltpu.VMEM((CHUNK, H), jnp.float32),
        pltpu.VMEM_SHARED((NACC, H), jnp.float32),
    ],
)(_counts_body)


def _mm_body(x_ref, w_ref, b_ref, o_ref):
    h = pl.program_id(1)
    x = x_ref[...]
    w = w_ref[...]
    o = lax.dot_general(x, w, (((1,), (1,)), ((), ())),
                        precision=lax.Precision.HIGHEST,
                        preferred_element_type=jnp.float32)
    o_ref[...] = o + b_ref[h, :][None, :]


def _matmul(data, W, b):
    return pl.pallas_call(
        _mm_body,
        grid=(GRID_I, 2),
        in_specs=[
            pl.BlockSpec((RB, D), lambda i, h: (i, 0)),
            pl.BlockSpec((H, D), lambda i, h: (h, 0)),
            pl.BlockSpec((2, H), lambda i, h: (0, 0)),
        ],
        out_specs=pl.BlockSpec((RB, H), lambda i, h: (h * GRID_I + i, 0)),
        out_shape=jax.ShapeDtypeStruct((2 * N, H), jnp.float32),
    )(data, W, b.reshape(2, H))


def _lap(x, g, cnt_ref):
    cnt = cnt_ref[...][:, 0:1]
    mask = cnt > 0.0
    invc = 1.0 / jnp.where(mask, cnt, 1.0)
    return jnp.where(mask, x - g * invc, 0.0)


def _combine_first_body(x_ref, g_ref, cnt_ref, y_ref, o_ref):
    x = x_ref[...]
    y = _lap(x, g_ref[...], cnt_ref)
    y_ref[...] = y
    o_ref[...] = y + x


def _combine_body(x_ref, g_ref, p_ref, a_ref, cnt_ref, y_ref, o_ref):
    y = 2.0 * _lap(x_ref[...], g_ref[...], cnt_ref) - p_ref[...]
    y_ref[...] = y
    o_ref[...] = a_ref[...] + y


def _final_body(x_ref, g_ref, p_ref, a_ref, cnt_ref, o_ref):
    y = 2.0 * _lap(x_ref[...], g_ref[...], cnt_ref) - p_ref[...]
    o_ref[...] = jnp.maximum(a_ref[...] + y, 0.0)


_cat_spec = pl.BlockSpec((RB, H), lambda i, h: (h * GRID_I + i, 0))
_cnt_spec = pl.BlockSpec((RB, H), lambda i, h: (i, 0))
_cat_struct = jax.ShapeDtypeStruct((2 * N, H), jnp.float32)


def _combine_first(x, g, cnt):
    return pl.pallas_call(
        _combine_first_body,
        grid=(GRID_I, 2),
        in_specs=[_cat_spec, _cat_spec, _cnt_spec],
        out_specs=[_cat_spec, _cat_spec],
        out_shape=[_cat_struct, _cat_struct],
    )(x, g, cnt)


def _combine(x, g, prev, out, cnt):
    return pl.pallas_call(
        _combine_body,
        grid=(GRID_I, 2),
        in_specs=[_cat_spec, _cat_spec, _cat_spec, _cat_spec, _cnt_spec],
        out_specs=[_cat_spec, _cat_spec],
        out_shape=[_cat_struct, _cat_struct],
    )(x, g, prev, out, cnt)


def _final(x, g, prev, out, cnt):
    return pl.pallas_call(
        _final_body,
        grid=(GRID_I, 2),
        in_specs=[_cat_spec, _cat_spec, _cat_spec, _cat_spec, _cnt_spec],
        out_specs=pl.BlockSpec((RB, H), lambda i, h: (i, h)),
        out_shape=jax.ShapeDtypeStruct((N, D), jnp.float32),
    )(x, g, prev, out, cnt)


def kernel(data, structure, W, b):
    src = structure[0]
    dst = structure[1]

    pad = E_PAD - E
    src_p = jnp.concatenate([src, jnp.zeros((pad,), jnp.int32)])
    dst_p = jnp.concatenate([dst, jnp.full((pad,), N, jnp.int32)])
    srcs2 = jnp.concatenate([src_p, src_p + N]).reshape(2 * (E_PAD // CHUNK), CHUNK)
    dst2 = dst_p.reshape(E_PAD // CHUNK, CHUNK)

    cnt = _counts(dst2)

    o2 = _matmul(data, W, b)
    g = _segsum(o2, srcs2, dst2)
    o1, out = _combine_first(o2, g, cnt)

    g = _segsum(o1, srcs2, dst2)
    o1b, out = _combine(o1, g, o2, out, cnt)

    g = _segsum(o1b, srcs2, dst2)
    o1c, out = _combine(o1b, g, o1, out, cnt)

    g = _segsum(o1c, srcs2, dst2)
    return _final(o1c, g, o1b, out, cnt)


# speedup vs baseline: 4.7177x; 1.0648x over previous
import functools

import jax
import jax.numpy as jnp
from jax import lax
from jax.experimental import pallas as pl
from jax.experimental.pallas import tpu as pltpu
from jax.experimental.pallas import tpu_sc as plsc

N = 10000
E = 160000
D = 256
H = 128
DEPTH = 3

NC = 2
NS = 16
CHUNK = 128
EPT = 10240
E_PAD = NS * EPT
NCH = EPT // CHUNK
NSEG = 5
SEGC = NCH // NSEG
NACC = 10240
RB = 1000
GRID_I = N // RB

_sc_mesh = plsc.VectorSubcoreMesh(core_axis_name="c", subcore_axis_name="s")


def _segsum_body(x_hbm, srcs_hbm, dst_hbm, g_hbm,
                 sbuf, dbuf, rb0, rb1, acc, sg0, sg1):
    c = lax.axis_index("c")
    s = lax.axis_index("s")
    rbs = (rb0, rb1)
    sgs = (sg0, sg1)

    @pl.loop(0, CHUNK)
    def _zero_rows(i):
        for k in range(H // 16):
            rb0[i, pl.ds(k * 16, 16)] = jnp.zeros((16,), jnp.float32)

    @pl.loop(0, NACC // NS // CHUNK)
    def _zero_acc(k):
        pltpu.sync_copy(rb0, acc.at[pl.ds(s * (NACC // NS) + k * CHUNK, CHUNK)])

    plsc.subcore_barrier()

    def _g_start(j, b):
        pltpu.async_copy(x_hbm.at[sbuf.at[j]], rbs[b], sgs[b])

    def _g_wait(b):
        pltpu.make_async_copy(x_hbm.at[sbuf.at[0]], rbs[b], sgs[b]).wait()

    def _s_sync(j, b):
        pltpu.sync_copy(rbs[b], acc.at[dbuf.at[j]], add=True)

    @pl.loop(0, NSEG)
    def _segment(seg):
        pltpu.sync_copy(
            srcs_hbm.at[pl.ds(c * (E_PAD // CHUNK) + s * NCH + seg * SEGC, SEGC)],
            sbuf)
        pltpu.sync_copy(dst_hbm.at[pl.ds(s * NCH + seg * SEGC, SEGC)], dbuf)

        _g_start(0, 0)
        _g_start(1, 1)

        @pl.loop(0, (SEGC - 2) // 2)
        def _main(jo):
            for b in range(2):
                j = jo * 2 + b
                _g_wait(b)
                _s_sync(j, b)
                _g_start(j + 2, b)

        _g_wait(0); _s_sync(SEGC - 2, 0)
        _g_wait(1); _s_sync(SEGC - 1, 1)

    plsc.subcore_barrier()

    @pl.when(s < NS - 1)
    def _flush_main():
        pltpu.sync_copy(acc.at[pl.ds(s * 632, 632)],
                        g_hbm.at[pl.ds(c * N + s * 632, 632)])

    @pl.when(s == NS - 1)
    def _flush_tail():
        pltpu.sync_copy(acc.at[pl.ds((NS - 1) * 632, N - (NS - 1) * 632)],
                        g_hbm.at[pl.ds(c * N + (NS - 1) * 632, N - (NS - 1) * 632)])


_segsum = functools.partial(
    pl.kernel,
    out_type=jax.ShapeDtypeStruct((2 * N, H), jnp.float32),
    mesh=_sc_mesh,
    scratch_types=[
        pltpu.VMEM((SEGC, CHUNK), jnp.int32),
        pltpu.VMEM((SEGC, CHUNK), jnp.int32),
        pltpu.VMEM((CHUNK, H), jnp.float32),
        pltpu.VMEM((CHUNK, H), jnp.float32),
        pltpu.VMEM_SHARED((NACC, H), jnp.float32),
    ] + [pltpu.SemaphoreType.DMA] * 2,
)(_segsum_body)


def _counts_body(dst_hbm, cnt_hbm, dbuf, buf, acc):
    c = lax.axis_index("c")
    s = lax.axis_index("s")

    @pl.when(c == 0)
    def _():
        pltpu.sync_copy(dst_hbm.at[pl.ds(s * NCH, NCH)], dbuf)

        @pl.loop(0, CHUNK)
        def _fill_zer(i):
            for k in range(H // 16):
                buf[i, pl.ds(k * 16, 16)] = jnp.zeros((16,), jnp.float32)

        @pl.loop(0, NACC // NS // CHUNK)
        def _zero_acc(k):
            pltpu.sync_copy(buf, acc.at[pl.ds(s * (NACC // NS) + k * CHUNK, CHUNK)])

        @pl.loop(0, CHUNK)
        def _fill_ones(i):
            for k in range(H // 16):
                buf[i, pl.ds(k * 16, 16)] = jnp.ones((16,), jnp.float32)

        plsc.subcore_barrier()

        @pl.loop(0, NCH)
        def _count(j):
            pltpu.sync_copy(buf, acc.at[dbuf.at[j]], add=True)

        plsc.subcore_barrier()
        pltpu.sync_copy(acc.at[pl.ds(s * (NACC // NS), NACC // NS)],
                        cnt_hbm.at[pl.ds(s * (NACC // NS), NACC // NS)])


_counts = functools.partial(
    pl.kernel,
    out_type=jax.ShapeDtypeStruct((NACC, H), jnp.float32),
    mesh=_sc_mesh,
    scratch_types=[
        pltpu.VMEM((NCH, CHUNK), jnp.int32),
        pltpu.VMEM((CHUNK, H), jnp.float32),
        pltpu.VMEM_SHARED((NACC, H), jnp.float32),
    ],
)(_counts_body)


def _mm_body(x_ref, w_ref, b_ref, o_ref):
    h = pl.program_id(1)
    x = x_ref[...]
    w = w_ref[...]
    o = lax.dot_general(x, w, (((1,), (1,)), ((), ())),
                        precision=lax.Precision.HIGHEST,
                        preferred_element_type=jnp.float32)
    o_ref[...] = o + b_ref[h, :][None, :]


def _matmul(data, W, b):
    return pl.pallas_call(
        _mm_body,
        grid=(GRID_I, 2),
        in_specs=[
            pl.BlockSpec((RB, D), lambda i, h: (i, 0)),
            pl.BlockSpec((H, D), lambda i, h: (h, 0)),
            pl.BlockSpec((2, H), lambda i, h: (0, 0)),
        ],
        out_specs=pl.BlockSpec((RB, H), lambda i, h: (h * GRID_I + i, 0)),
        out_shape=jax.ShapeDtypeStruct((2 * N, H), jnp.float32),
    )(data, W, b.reshape(2, H))


def _lap(x, g, cnt_ref):
    cnt = cnt_ref[...][:, 0:1]
    mask = cnt > 0.0
    invc = 1.0 / jnp.where(mask, cnt, 1.0)
    return jnp.where(mask, x - g * invc, 0.0)


def _combine_first_body(x_ref, g_ref, cnt_ref, y_ref, o_ref):
    x = x_ref[...]
    y = _lap(x, g_ref[...], cnt_ref)
    y_ref[...] = y
    o_ref[...] = y + x


def _combine_body(x_ref, g_ref, p_ref, a_ref, cnt_ref, y_ref, o_ref):
    y = 2.0 * _lap(x_ref[...], g_ref[...], cnt_ref) - p_ref[...]
    y_ref[...] = y
    o_ref[...] = a_ref[...] + y


def _final_body(x_ref, g_ref, p_ref, a_ref, cnt_ref, o_ref):
    y = 2.0 * _lap(x_ref[...], g_ref[...], cnt_ref) - p_ref[...]
    o_ref[...] = jnp.maximum(a_ref[...] + y, 0.0)


_cat_spec = pl.BlockSpec((RB, H), lambda i, h: (h * GRID_I + i, 0))
_cnt_spec = pl.BlockSpec((RB, H), lambda i, h: (i, 0))
_cat_struct = jax.ShapeDtypeStruct((2 * N, H), jnp.float32)


def _combine_first(x, g, cnt):
    return pl.pallas_call(
        _combine_first_body,
        grid=(GRID_I, 2),
        in_specs=[_cat_spec, _cat_spec, _cnt_spec],
        out_specs=[_cat_spec, _cat_spec],
        out_shape=[_cat_struct, _cat_struct],
    )(x, g, cnt)


def _combine(x, g, prev, out, cnt):
    return pl.pallas_call(
        _combine_body,
        grid=(GRID_I, 2),
        in_specs=[_cat_spec, _cat_spec, _cat_spec, _cat_spec, _cnt_spec],
        out_specs=[_cat_spec, _cat_spec],
        out_shape=[_cat_struct, _cat_struct],
    )(x, g, prev, out, cnt)


def _final(x, g, prev, out, cnt):
    return pl.pallas_call(
        _final_body,
        grid=(GRID_I, 2),
        in_specs=[_cat_spec, _cat_spec, _cat_spec, _cat_spec, _cnt_spec],
        out_specs=pl.BlockSpec((RB, H), lambda i, h: (i, h)),
        out_shape=jax.ShapeDtypeStruct((N, D), jnp.float32),
    )(x, g, prev, out, cnt)


def kernel(data, structure, W, b):
    src = structure[0]
    dst = structure[1]

    pad = E_PAD - E
    src_p = jnp.concatenate([src, jnp.zeros((pad,), jnp.int32)])
    dst_p = jnp.concatenate([dst, jnp.full((pad,), N, jnp.int32)])
    srcs2 = jnp.concatenate([src_p, src_p + N]).reshape(2 * (E_PAD // CHUNK), CHUNK)
    dst2 = dst_p.reshape(E_PAD // CHUNK, CHUNK)

    cnt = _counts(dst2)

    o2 = _matmul(data, W, b)
    g = _segsum(o2, srcs2, dst2)
    o1, out = _combine_first(o2, g, cnt)

    g = _segsum(o1, srcs2, dst2)
    o1b, out = _combine(o1, g, o2, out, cnt)

    g = _segsum(o1b, srcs2, dst2)
    o1c, out = _combine(o1b, g, o1, out, cnt)

    g = _segsum(o1c, srcs2, dst2)
    return _final(o1c, g, o1b, out, cnt)
